# baseline (speedup 1.0000x reference)
import functools

import jax
import jax.numpy as jnp
from jax import lax
from jax.experimental import pallas as pl
from jax.experimental.pallas import tpu as pltpu
from jax.experimental.pallas import tpu_sc as plsc

_N = 10000
_NP = 10240
_E = 320000
_D = 128
_P = 16384

_NC = 2
_NS = 16
_NW = _NC * _NS
_L = 16

_HD = _D // 2
_EC = 80
_EPT = _E // _NS
_NCH = _EPT // _EC
_EPW = _E // _NW
_DC = 2000
_NDC = _EPW // _DC
_RPT = _NP // _NS
_PPW = _P // _NW


def _zero16():
    return jnp.zeros((_L,), jnp.float32)


def _sc_agg_body(with_deg, h_lo_hbm, h_hi_hbm, src_hbm, dst_hbm, *rest):
    if with_deg:
        (lo_out, hi_out, dega_out, degb_out, h_s, acc, deg_s,
         idx_sa, idx_sb, idx_da, idx_db,
         idx_g0, idx_g1, idx_g2, idx_g3, idx_g4,
         rows_a, rows_b, ones_v, sem_a, sem_b) = rest
        idx_gs = (idx_g0, idx_g1, idx_g2, idx_g3, idx_g4)
    else:
        (lo_out, hi_out, h_s, acc, deg_s,
         idx_sa, idx_sb, idx_da, idx_db,
         idx_g0, idx_g1, idx_g2, idx_g3, idx_g4,
         rows_a, rows_b, ones_v, sem_a, sem_b) = rest
        idx_gs = ()
    c = lax.axis_index("c")
    s = lax.axis_index("s")
    wid = s * _NC + c
    r0 = s * _RPT

    def zrow(i, _):
        for k in range(_HD // _L):
            rows_a[i, pl.ds(k * _L, _L)] = _zero16()
        return 0
    lax.fori_loop(0, _EC, zrow, 0)

    def zacc(i, _):
        pltpu.sync_copy(rows_a, acc.at[pl.ds(r0 + i * _EC, _EC)])
        return 0
    lax.fori_loop(0, _RPT // _EC, zacc, 0)

    @pl.when(c == 0)
    def _():
        pltpu.sync_copy(h_lo_hbm.at[pl.ds(r0, _RPT)], h_s.at[pl.ds(r0, _RPT)])

    @pl.when(c == 1)
    def _():
        pltpu.sync_copy(h_hi_hbm.at[pl.ds(r0, _RPT)], h_s.at[pl.ds(r0, _RPT)])

    def ofill(i, val):
        ones_v[pl.ds(i * _L, _L)] = jnp.full((_L,), val, jnp.float32)
        return val
    if with_deg:
        lax.fori_loop(0, _DC // _L, ofill, 0.0)
        pltpu.sync_copy(ones_v.at[pl.ds(0, _RPT)], deg_s.at[pl.ds(r0, _RPT)])
        lax.fori_loop(0, _DC // _L, ofill, 1.0)

    plsc.subcore_barrier()

    if with_deg:
        for j, g in enumerate(idx_gs):
            pltpu.sync_copy(dst_hbm.at[pl.ds(wid * _EPW + j * _DC, _DC)], g)
        for g in idx_gs:
            pltpu.sync_copy(ones_v, deg_s.at[g], add=True)

    def gstart(chunk, idxbuf, rbuf, sem):
        base = s * _EPT + chunk * _EC
        pltpu.sync_copy(src_hbm.at[pl.ds(base, _EC)], idxbuf)
        pltpu.async_copy(h_s.at[idxbuf], rbuf, sem)

    def gwait(idxbuf, rbuf, sem):
        pltpu.make_async_copy(h_s.at[idxbuf], rbuf, sem).wait()

    def scat(chunk, idxbuf, rbuf):
        base = s * _EPT + chunk * _EC
        pltpu.sync_copy(dst_hbm.at[pl.ds(base, _EC)], idxbuf)
        pltpu.sync_copy(rbuf, acc.at[idxbuf], add=True)

    gstart(0, idx_sa, rows_a, sem_a)

    def body(i, _):
        a = 2 * i
        gstart(a + 1, idx_sb, rows_b, sem_b)
        gwait(idx_sa, rows_a, sem_a)
        scat(a, idx_da, rows_a)
        gstart(lax.rem(a + 2, _NCH), idx_sa, rows_a, sem_a)
        gwait(idx_sb, rows_b, sem_b)
        scat(a + 1, idx_db, rows_b)
        return 0
    lax.fori_loop(0, _NCH // 2, body, 0)
    gwait(idx_sa, rows_a, sem_a)

    plsc.subcore_barrier()

    @pl.when(c == 0)
    def _():
        pltpu.sync_copy(acc.at[pl.ds(r0, _RPT)], lo_out.at[pl.ds(r0, _RPT)])

    @pl.when(c == 1)
    def _():
        pltpu.sync_copy(acc.at[pl.ds(r0, _RPT)], hi_out.at[pl.ds(r0, _RPT)])

    if with_deg:
        @pl.when(c == 0)
        def _():
            pltpu.sync_copy(deg_s.at[pl.ds(r0, _RPT)],
                            dega_out.at[pl.ds(r0, _RPT)])

        @pl.when(c == 1)
        def _():
            pltpu.sync_copy(deg_s.at[pl.ds(r0, _RPT)],
                            degb_out.at[pl.ds(r0, _RPT)])


def _agg_scratch():
    return [
        pltpu.VMEM_SHARED((_NP, _HD), jnp.float32),
        pltpu.VMEM_SHARED((_NP, _HD), jnp.float32),
        pltpu.VMEM_SHARED((_NP,), jnp.float32),
        pltpu.VMEM((_EC,), jnp.int32),
        pltpu.VMEM((_EC,), jnp.int32),
        pltpu.VMEM((_EC,), jnp.int32),
        pltpu.VMEM((_EC,), jnp.int32),
        pltpu.VMEM((_DC,), jnp.int32),
        pltpu.VMEM((_DC,), jnp.int32),
        pltpu.VMEM((_DC,), jnp.int32),
        pltpu.VMEM((_DC,), jnp.int32),
        pltpu.VMEM((_DC,), jnp.int32),
        pltpu.VMEM((_EC, _HD), jnp.float32),
        pltpu.VMEM((_EC, _HD), jnp.float32),
        pltpu.VMEM((_DC,), jnp.float32),
        pltpu.SemaphoreType.DMA,
        pltpu.SemaphoreType.DMA,
    ]


_half_shape = jax.ShapeDtypeStruct((_NP, _HD), jnp.float32)

_sc_agg_deg = functools.partial(
    pl.kernel,
    out_type=[
        _half_shape,
        _half_shape,
        jax.ShapeDtypeStruct((_NP,), jnp.float32),
        jax.ShapeDtypeStruct((_NP,), jnp.float32),
    ],
    mesh=plsc.VectorSubcoreMesh(core_axis_name="c", subcore_axis_name="s"),
    scratch_types=_agg_scratch(),
)(functools.partial(_sc_agg_body, True))

_sc_agg_nodeg = functools.partial(
    pl.kernel,
    out_type=[_half_shape, _half_shape],
    mesh=plsc.VectorSubcoreMesh(core_axis_name="c", subcore_axis_name="s"),
    scratch_types=_agg_scratch(),
)(functools.partial(_sc_agg_body, False))


def _sc_gather_body(h_hbm, ps_hbm, pd_hbm, ns_hbm, nd_hbm, bias_hbm,
                    sp_out, dp_out, sn_out, dn_out, bdiff_out,
                    rows, psv, pdv, nsv, ndv,
                    pbs_v, pbd_v, nbs_v, nbd_v, bd_v):
    c = lax.axis_index("c")
    s = lax.axis_index("s")
    wid = s * _NC + c
    base = wid * _PPW

    pltpu.sync_copy(ps_hbm.at[pl.ds(base, _PPW)], psv)
    pltpu.sync_copy(pd_hbm.at[pl.ds(base, _PPW)], pdv)
    pltpu.sync_copy(ns_hbm.at[pl.ds(base, _PPW)], nsv)
    pltpu.sync_copy(nd_hbm.at[pl.ds(base, _PPW)], ndv)

    pltpu.sync_copy(bias_hbm.at[psv], pbs_v)
    pltpu.sync_copy(bias_hbm.at[pdv], pbd_v)
    pltpu.sync_copy(bias_hbm.at[nsv], nbs_v)
    pltpu.sync_copy(bias_hbm.at[ndv], nbd_v)

    def comb(t, _):
        o = t * _L
        bd_v[pl.ds(o, _L)] = (nbs_v[pl.ds(o, _L)] + nbd_v[pl.ds(o, _L)]
                              - pbs_v[pl.ds(o, _L)] - pbd_v[pl.ds(o, _L)])
        return 0
    lax.fori_loop(0, _PPW // _L, comb, 0)
    pltpu.sync_copy(bd_v, bdiff_out.at[pl.ds(base, _PPW)])

    for idx_v, out in ((psv, sp_out), (pdv, dp_out),
                       (nsv, sn_out), (ndv, dn_out)):
        pltpu.sync_copy(h_hbm.at[idx_v], rows)
        pltpu.sync_copy(rows, out.at[pl.ds(base, _PPW)])


_sc_gather = functools.partial(
    pl.kernel,
    out_type=[
        jax.ShapeDtypeStruct((_P, _D), jnp.float32),
        jax.ShapeDtypeStruct((_P, _D), jnp.float32),
        jax.ShapeDtypeStruct((_P, _D), jnp.float32),
        jax.ShapeDtypeStruct((_P, _D), jnp.float32),
        jax.ShapeDtypeStruct((_P,), jnp.float32),
    ],
    mesh=plsc.VectorSubcoreMesh(core_axis_name="c", subcore_axis_name="s"),
    scratch_types=[
        pltpu.VMEM((_PPW, _D), jnp.float32),
        pltpu.VMEM((_PPW,), jnp.int32),
        pltpu.VMEM((_PPW,), jnp.int32),
        pltpu.VMEM((_PPW,), jnp.int32),
        pltpu.VMEM((_PPW,), jnp.int32),
        pltpu.VMEM((_PPW,), jnp.float32),
        pltpu.VMEM((_PPW,), jnp.float32),
        pltpu.VMEM((_PPW,), jnp.float32),
        pltpu.VMEM((_PPW,), jnp.float32),
        pltpu.VMEM((_PPW,), jnp.float32),
    ],
)(_sc_gather_body)


_BR = 1024


def _dot(a, b):
    return jnp.dot(a, b, preferred_element_type=jnp.float32,
                   precision=lax.Precision.HIGHEST)


def _tc1_body(x_ref, wp_ref, ws1_ref, lo_ref, hi_ref, s1_ref):
    h0 = _dot(x_ref[...], wp_ref[...])
    lo_ref[...] = h0[:, :_HD]
    hi_ref[...] = h0[:, _HD:]
    s1_ref[...] = _dot(h0, ws1_ref[...])


def _neigh_relu(selfp_ref, alo_ref, ahi_ref, dega_ref, degb_ref, wn_ref,
                bias_ref):
    scale = 1.0 / jnp.maximum(dega_ref[...] + degb_ref[...], 1.0)
    wn = wn_ref[...]
    neigh = (_dot(alo_ref[...] * scale, wn[:_HD, :])
             + _dot(ahi_ref[...] * scale, wn[_HD:, :]))
    return jnp.maximum(selfp_ref[...] + neigh + bias_ref[...], 0.0)


def _tc_layer_body(selfp_ref, alo_ref, ahi_ref, dega_ref, degb_ref, wn_ref,
                   bias_ref, ws_next_ref, lo_ref, hi_ref, snext_ref):
    h = _neigh_relu(selfp_ref, alo_ref, ahi_ref, dega_ref, degb_ref,
                    wn_ref, bias_ref)
    lo_ref[...] = h[:, :_HD]
    hi_ref[...] = h[:, _HD:]
    snext_ref[...] = _dot(h, ws_next_ref[...])


def _tc_final_body(selfp_ref, alo_ref, ahi_ref, dega_ref, degb_ref, wn_ref,
                   bias_ref, h0lo_ref, h0hi_ref, out_ref):
    h = _neigh_relu(selfp_ref, alo_ref, ahi_ref, dega_ref, degb_ref,
                    wn_ref, bias_ref)
    h0 = jnp.concatenate([h0lo_ref[...], h0hi_ref[...]], axis=1)
    out_ref[...] = h0 + h


def _tc_score_body(sp_ref, dp_ref, sn_ref, dn_ref, bd_ref, out_ref):
    posdot = jnp.sum(sp_ref[...] * dp_ref[...], axis=1, keepdims=True)
    negdot = jnp.sum(sn_ref[...] * dn_ref[...], axis=1, keepdims=True)
    out_ref[...] = jnp.maximum(negdot - posdot + bd_ref[...] + 1.0, 0.0)


def _row_spec():
    return pl.BlockSpec((_BR, _D), lambda i: (i, 0))


def _half_spec():
    return pl.BlockSpec((_BR, _HD), lambda i: (i, 0))


def _w_spec():
    return pl.BlockSpec((_D, _D), lambda i: (0, 0))


def _b_spec():
    return pl.BlockSpec((1, _D), lambda i: (0, 0))


def _deg_spec():
    return pl.BlockSpec((_BR, 1), lambda i: (i, 0))


_GRID = (_NP // _BR,)

_full_shape = jax.ShapeDtypeStruct((_NP, _D), jnp.float32)

_tc1 = pl.pallas_call(
    _tc1_body,
    grid=_GRID,
    in_specs=[_row_spec(), _w_spec(), _w_spec()],
    out_specs=[_half_spec(), _half_spec(), _row_spec()],
    out_shape=[_half_shape, _half_shape, _full_shape],
)

_tc_layer = pl.pallas_call(
    _tc_layer_body,
    grid=_GRID,
    in_specs=[_row_spec(), _half_spec(), _half_spec(), _deg_spec(),
              _deg_spec(), _w_spec(), _b_spec(), _w_spec()],
    out_specs=[_half_spec(), _half_spec(), _row_spec()],
    out_shape=[_half_shape, _half_shape, _full_shape],
)

_tc_final = pl.pallas_call(
    _tc_final_body,
    grid=_GRID,
    in_specs=[_row_spec(), _half_spec(), _half_spec(), _deg_spec(),
              _deg_spec(), _w_spec(), _b_spec(), _half_spec(),
              _half_spec()],
    out_specs=_row_spec(),
    out_shape=_full_shape,
)

_tc_score = pl.pallas_call(
    _tc_score_body,
    grid=(_P // _BR,),
    in_specs=[_row_spec(), _row_spec(), _row_spec(), _row_spec(),
              _deg_spec()],
    out_specs=_deg_spec(),
    out_shape=jax.ShapeDtypeStruct((_P, 1), jnp.float32),
)


def kernel(x, edge_index, pos_edges, neg_edges, W_proj, W_self1, W_neigh1,
           b1, W_self2, W_neigh2, b2, item_bias):
    xp = jnp.pad(x, ((0, _NP - _N), (0, 0)))
    src = edge_index[0]
    dst = edge_index[1]

    h0lo, h0hi, s1 = _tc1(xp, W_proj, W_self1)
    agg1lo, agg1hi, dega1, degb1 = _sc_agg_deg(h0lo, h0hi, src, dst)
    dega = dega1[:, None]
    degb = degb1[:, None]
    h1lo, h1hi, s2 = _tc_layer(s1, agg1lo, agg1hi, dega, degb, W_neigh1,
                               b1.reshape(1, _D), W_self2)
    agg2lo, agg2hi = _sc_agg_nodeg(h1lo, h1hi, src, dst)
    h_item = _tc_final(s2, agg2lo, agg2hi, dega, degb, W_neigh2,
                       b2.reshape(1, _D), h0lo, h0hi)
    sp, dp, sn, dn, bdiff = _sc_gather(h_item, pos_edges[0], pos_edges[1],
                                       neg_edges[0], neg_edges[1], item_bias)
    return _tc_score(sp, dp, sn, dn, bdiff[:, None]).reshape(_P)


# speedup vs baseline: 1.4332x; 1.4332x over previous
import functools

import jax
import jax.numpy as jnp
from jax import lax
from jax.experimental import pallas as pl
from jax.experimental.pallas import tpu as pltpu
from jax.experimental.pallas import tpu_sc as plsc

_N = 10000
_NP = 10240
_E = 320000
_D = 128
_P = 16384

_NC = 2
_NS = 16
_NW = _NC * _NS
_L = 16

_HD = _D // 2
_EC = 80
_EPW = _E // _NW
_NCH = _EPW // _EC
_DC = 2000
_NDC = _EPW // _DC
_RPT = _NP // _NS
_PPW = _P // _NW


def _zero16():
    return jnp.zeros((_L,), jnp.float32)


def _sc_agg_body(with_deg, h_hbm, src_hbm, dst_hbm, *rest):
    if with_deg:
        (agg_out, dega_out, degb_out, acc, deg_s,
         idx_sa, idx_sb, idx_da, idx_db,
         idx_g0, idx_g1, idx_g2, idx_g3, idx_g4,
         rows_a, rows_b, ones_v, sem_a, sem_b) = rest
        idx_gs = (idx_g0, idx_g1, idx_g2, idx_g3, idx_g4)
    else:
        (agg_out, acc, deg_s, idx_sa, idx_sb, idx_da, idx_db,
         idx_g0, idx_g1, idx_g2, idx_g3, idx_g4,
         rows_a, rows_b, ones_v, sem_a, sem_b) = rest
        idx_gs = ()
    c = lax.axis_index("c")
    s = lax.axis_index("s")
    wid = s * _NC + c
    r0 = s * _RPT

    def zrow(i, _):
        for k in range(_D // _L):
            rows_a[i, pl.ds(k * _L, _L)] = _zero16()
        return 0
    lax.fori_loop(0, _EC, zrow, 0)

    def zacc(i, _):
        pltpu.sync_copy(rows_a, acc.at[pl.ds(r0 + i * _EC, _EC)])
        return 0
    lax.fori_loop(0, _RPT // _EC, zacc, 0)

    def ofill(i, val):
        ones_v[pl.ds(i * _L, _L)] = jnp.full((_L,), val, jnp.float32)
        return val
    if with_deg:
        lax.fori_loop(0, _DC // _L, ofill, 0.0)
        pltpu.sync_copy(ones_v.at[pl.ds(0, _RPT)], deg_s.at[pl.ds(r0, _RPT)])
        lax.fori_loop(0, _DC // _L, ofill, 1.0)

    plsc.subcore_barrier()

    if with_deg:
        for j, g in enumerate(idx_gs):
            pltpu.sync_copy(dst_hbm.at[pl.ds(wid * _EPW + j * _DC, _DC)], g)
        for g in idx_gs:
            pltpu.sync_copy(ones_v, deg_s.at[g], add=True)

    def gstart(chunk, idxbuf, rbuf, sem):
        base = wid * _EPW + chunk * _EC
        pltpu.sync_copy(src_hbm.at[pl.ds(base, _EC)], idxbuf)
        pltpu.async_copy(h_hbm.at[idxbuf], rbuf, sem)

    def gwait(idxbuf, rbuf, sem):
        pltpu.make_async_copy(h_hbm.at[idxbuf], rbuf, sem).wait()

    def scat(chunk, idxbuf, rbuf):
        base = wid * _EPW + chunk * _EC
        pltpu.sync_copy(dst_hbm.at[pl.ds(base, _EC)], idxbuf)
        pltpu.sync_copy(rbuf, acc.at[idxbuf], add=True)

    gstart(0, idx_sa, rows_a, sem_a)

    def body(i, _):
        a = 2 * i
        gstart(a + 1, idx_sb, rows_b, sem_b)
        gwait(idx_sa, rows_a, sem_a)
        scat(a, idx_da, rows_a)
        gstart(a + 2, idx_sa, rows_a, sem_a)
        gwait(idx_sb, rows_b, sem_b)
        scat(a + 1, idx_db, rows_b)
        return 0
    lax.fori_loop(0, (_NCH - 1) // 2, body, 0)
    gwait(idx_sa, rows_a, sem_a)
    scat(_NCH - 1, idx_da, rows_a)

    plsc.subcore_barrier()

    pltpu.sync_copy(acc.at[pl.ds(r0, _RPT)], agg_out.at[c, pl.ds(r0, _RPT)])

    if with_deg:
        @pl.when(c == 0)
        def _():
            pltpu.sync_copy(deg_s.at[pl.ds(r0, _RPT)],
                            dega_out.at[pl.ds(r0, _RPT)])

        @pl.when(c == 1)
        def _():
            pltpu.sync_copy(deg_s.at[pl.ds(r0, _RPT)],
                            degb_out.at[pl.ds(r0, _RPT)])


def _agg_scratch():
    return [
        pltpu.VMEM_SHARED((_NP, _D), jnp.float32),
        pltpu.VMEM_SHARED((_NP,), jnp.float32),
        pltpu.VMEM((_EC,), jnp.int32),
        pltpu.VMEM((_EC,), jnp.int32),
        pltpu.VMEM((_EC,), jnp.int32),
        pltpu.VMEM((_EC,), jnp.int32),
        pltpu.VMEM((_DC,), jnp.int32),
        pltpu.VMEM((_DC,), jnp.int32),
        pltpu.VMEM((_DC,), jnp.int32),
        pltpu.VMEM((_DC,), jnp.int32),
        pltpu.VMEM((_DC,), jnp.int32),
        pltpu.VMEM((_EC, _D), jnp.float32),
        pltpu.VMEM((_EC, _D), jnp.float32),
        pltpu.VMEM((_DC,), jnp.float32),
        pltpu.SemaphoreType.DMA,
        pltpu.SemaphoreType.DMA,
    ]


_half_shape = jax.ShapeDtypeStruct((_NP, _HD), jnp.float32)

_sc_agg_deg = functools.partial(
    pl.kernel,
    out_type=[
        jax.ShapeDtypeStruct((_NC, _NP, _D), jnp.float32),
        jax.ShapeDtypeStruct((_NP,), jnp.float32),
        jax.ShapeDtypeStruct((_NP,), jnp.float32),
    ],
    mesh=plsc.VectorSubcoreMesh(core_axis_name="c", subcore_axis_name="s"),
    scratch_types=_agg_scratch(),
)(functools.partial(_sc_agg_body, True))

_sc_agg_nodeg = functools.partial(
    pl.kernel,
    out_type=jax.ShapeDtypeStruct((_NC, _NP, _D), jnp.float32),
    mesh=plsc.VectorSubcoreMesh(core_axis_name="c", subcore_axis_name="s"),
    scratch_types=_agg_scratch(),
)(functools.partial(_sc_agg_body, False))


def _sc_gather_body(h_hbm, ps_hbm, pd_hbm, ns_hbm, nd_hbm, bias_hbm,
                    sp_out, dp_out, sn_out, dn_out, bdiff_out,
                    rows, psv, pdv, nsv, ndv,
                    pbs_v, pbd_v, nbs_v, nbd_v, bd_v):
    c = lax.axis_index("c")
    s = lax.axis_index("s")
    wid = s * _NC + c
    base = wid * _PPW

    pltpu.sync_copy(ps_hbm.at[pl.ds(base, _PPW)], psv)
    pltpu.sync_copy(pd_hbm.at[pl.ds(base, _PPW)], pdv)
    pltpu.sync_copy(ns_hbm.at[pl.ds(base, _PPW)], nsv)
    pltpu.sync_copy(nd_hbm.at[pl.ds(base, _PPW)], ndv)

    pltpu.sync_copy(bias_hbm.at[psv], pbs_v)
    pltpu.sync_copy(bias_hbm.at[pdv], pbd_v)
    pltpu.sync_copy(bias_hbm.at[nsv], nbs_v)
    pltpu.sync_copy(bias_hbm.at[ndv], nbd_v)

    def comb(t, _):
        o = t * _L
        bd_v[pl.ds(o, _L)] = (nbs_v[pl.ds(o, _L)] + nbd_v[pl.ds(o, _L)]
                              - pbs_v[pl.ds(o, _L)] - pbd_v[pl.ds(o, _L)])
        return 0
    lax.fori_loop(0, _PPW // _L, comb, 0)
    pltpu.sync_copy(bd_v, bdiff_out.at[pl.ds(base, _PPW)])

    for idx_v, out in ((psv, sp_out), (pdv, dp_out),
                       (nsv, sn_out), (ndv, dn_out)):
        pltpu.sync_copy(h_hbm.at[idx_v], rows)
        pltpu.sync_copy(rows, out.at[pl.ds(base, _PPW)])


_sc_gather = functools.partial(
    pl.kernel,
    out_type=[
        jax.ShapeDtypeStruct((_P, _D), jnp.float32),
        jax.ShapeDtypeStruct((_P, _D), jnp.float32),
        jax.ShapeDtypeStruct((_P, _D), jnp.float32),
        jax.ShapeDtypeStruct((_P, _D), jnp.float32),
        jax.ShapeDtypeStruct((_P,), jnp.float32),
    ],
    mesh=plsc.VectorSubcoreMesh(core_axis_name="c", subcore_axis_name="s"),
    scratch_types=[
        pltpu.VMEM((_PPW, _D), jnp.float32),
        pltpu.VMEM((_PPW,), jnp.int32),
        pltpu.VMEM((_PPW,), jnp.int32),
        pltpu.VMEM((_PPW,), jnp.int32),
        pltpu.VMEM((_PPW,), jnp.int32),
        pltpu.VMEM((_PPW,), jnp.float32),
        pltpu.VMEM((_PPW,), jnp.float32),
        pltpu.VMEM((_PPW,), jnp.float32),
        pltpu.VMEM((_PPW,), jnp.float32),
        pltpu.VMEM((_PPW,), jnp.float32),
    ],
)(_sc_gather_body)


_BR = 1024


def _dot(a, b):
    return jnp.dot(a, b, preferred_element_type=jnp.float32,
                   precision=lax.Precision.HIGHEST)


def _tc1_body(x_ref, wp_ref, ws1_ref, h0_ref, s1_ref):
    h0 = _dot(x_ref[...], wp_ref[...])
    h0_ref[...] = h0
    s1_ref[...] = _dot(h0, ws1_ref[...])


def _neigh_relu(selfp_ref, a_ref, b_ref, dega_ref, degb_ref, wn_ref,
                bias_ref):
    scale = 1.0 / jnp.maximum(dega_ref[...] + degb_ref[...], 1.0)
    agg = (a_ref[...] + b_ref[...]) * scale
    neigh = _dot(agg, wn_ref[...])
    return jnp.maximum(selfp_ref[...] + neigh + bias_ref[...], 0.0)


def _tc_layer_body(selfp_ref, a_ref, b_ref, dega_ref, degb_ref, wn_ref,
                   bias_ref, ws_next_ref, h_ref, snext_ref):
    h = _neigh_relu(selfp_ref, a_ref, b_ref, dega_ref, degb_ref,
                    wn_ref, bias_ref)
    h_ref[...] = h
    snext_ref[...] = _dot(h, ws_next_ref[...])


def _tc_final_body(selfp_ref, a_ref, b_ref, dega_ref, degb_ref, wn_ref,
                   bias_ref, h0_ref, out_ref):
    h = _neigh_relu(selfp_ref, a_ref, b_ref, dega_ref, degb_ref,
                    wn_ref, bias_ref)
    out_ref[...] = h0_ref[...] + h


def _tc_score_body(sp_ref, dp_ref, sn_ref, dn_ref, bd_ref, out_ref):
    posdot = jnp.sum(sp_ref[...] * dp_ref[...], axis=1, keepdims=True)
    negdot = jnp.sum(sn_ref[...] * dn_ref[...], axis=1, keepdims=True)
    out_ref[...] = jnp.maximum(negdot - posdot + bd_ref[...] + 1.0, 0.0)


def _row_spec():
    return pl.BlockSpec((_BR, _D), lambda i: (i, 0))


def _half_spec():
    return pl.BlockSpec((_BR, _HD), lambda i: (i, 0))


def _w_spec():
    return pl.BlockSpec((_D, _D), lambda i: (0, 0))


def _b_spec():
    return pl.BlockSpec((1, _D), lambda i: (0, 0))


def _deg_spec():
    return pl.BlockSpec((_BR, 1), lambda i: (i, 0))


_GRID = (_NP // _BR,)

_full_shape = jax.ShapeDtypeStruct((_NP, _D), jnp.float32)

_tc1 = pl.pallas_call(
    _tc1_body,
    grid=_GRID,
    in_specs=[_row_spec(), _w_spec(), _w_spec()],
    out_specs=[_row_spec(), _row_spec()],
    out_shape=[_full_shape, _full_shape],
)

_tc_layer = pl.pallas_call(
    _tc_layer_body,
    grid=_GRID,
    in_specs=[_row_spec(), _row_spec(), _row_spec(), _deg_spec(),
              _deg_spec(), _w_spec(), _b_spec(), _w_spec()],
    out_specs=[_row_spec(), _row_spec()],
    out_shape=[_full_shape, _full_shape],
)

_tc_final = pl.pallas_call(
    _tc_final_body,
    grid=_GRID,
    in_specs=[_row_spec(), _row_spec(), _row_spec(), _deg_spec(),
              _deg_spec(), _w_spec(), _b_spec(), _row_spec()],
    out_specs=_row_spec(),
    out_shape=_full_shape,
)

_tc_score = pl.pallas_call(
    _tc_score_body,
    grid=(_P // _BR,),
    in_specs=[_row_spec(), _row_spec(), _row_spec(), _row_spec(),
              _deg_spec()],
    out_specs=_deg_spec(),
    out_shape=jax.ShapeDtypeStruct((_P, 1), jnp.float32),
)


def kernel(x, edge_index, pos_edges, neg_edges, W_proj, W_self1, W_neigh1,
           b1, W_self2, W_neigh2, b2, item_bias):
    xp = jnp.pad(x, ((0, _NP - _N), (0, 0)))
    src = edge_index[0]
    dst = edge_index[1]

    h0, s1 = _tc1(xp, W_proj, W_self1)
    agg1, dega1, degb1 = _sc_agg_deg(h0, src, dst)
    dega = dega1[:, None]
    degb = degb1[:, None]
    h1, s2 = _tc_layer(s1, agg1[0], agg1[1], dega, degb, W_neigh1,
                       b1.reshape(1, _D), W_self2)
    agg2 = _sc_agg_nodeg(h1, src, dst)
    h_item = _tc_final(s2, agg2[0], agg2[1], dega, degb, W_neigh2,
                       b2.reshape(1, _D), h0)
    sp, dp, sn, dn, bdiff = _sc_gather(h_item, pos_edges[0], pos_edges[1],
                                       neg_edges[0], neg_edges[1], item_bias)
    return _tc_score(sp, dp, sn, dn, bdiff[:, None]).reshape(_P)


# speedup vs baseline: 1.4740x; 1.0284x over previous
import functools

import jax
import jax.numpy as jnp
from jax import lax
from jax.experimental import pallas as pl
from jax.experimental.pallas import tpu as pltpu
from jax.experimental.pallas import tpu_sc as plsc

_N = 10000
_NP = 10240
_E = 320000
_D = 128
_P = 16384

_NC = 2
_NS = 16
_NW = _NC * _NS
_L = 16

_HD = _D // 2
_EC = 80
_EPW = _E // _NW
_NCH = _EPW // _EC
_DC = 2000
_NDC = _EPW // _DC
_RPT = _NP // _NS
_PPW = _P // _NW


def _zero16():
    return jnp.zeros((_L,), jnp.float32)


def _sc_agg_body(with_deg, h_hbm, src_hbm, dst_hbm, *rest):
    if with_deg:
        (agg_out, dega_out, degb_out, acc, deg_s,
         idx_sa, idx_sb, idx_da, idx_db,
         idx_g0, idx_g1, idx_g2, idx_g3, idx_g4,
         rows_a, rows_b, ones_v, sem_a, sem_b) = rest
        idx_gs = (idx_g0, idx_g1, idx_g2, idx_g3, idx_g4)
    else:
        (agg_out, acc, deg_s, idx_sa, idx_sb, idx_da, idx_db,
         idx_g0, idx_g1, idx_g2, idx_g3, idx_g4,
         rows_a, rows_b, ones_v, sem_a, sem_b) = rest
        idx_gs = ()
    c = lax.axis_index("c")
    s = lax.axis_index("s")
    wid = s * _NC + c
    r0 = s * _RPT

    def zrow(i, _):
        for k in range(_D // _L):
            rows_a[i, pl.ds(k * _L, _L)] = _zero16()
        return 0
    lax.fori_loop(0, _EC, zrow, 0)

    def zacc(i, _):
        pltpu.sync_copy(rows_a, acc.at[pl.ds(r0 + i * _EC, _EC)])
        return 0
    lax.fori_loop(0, _RPT // _EC, zacc, 0)

    def ofill(i, val):
        ones_v[pl.ds(i * _L, _L)] = jnp.full((_L,), val, jnp.float32)
        return val
    if with_deg:
        lax.fori_loop(0, _DC // _L, ofill, 0.0)
        pltpu.sync_copy(ones_v.at[pl.ds(0, _RPT)], deg_s.at[pl.ds(r0, _RPT)])
        lax.fori_loop(0, _DC // _L, ofill, 1.0)

    plsc.subcore_barrier()

    if with_deg:
        for j, g in enumerate(idx_gs):
            pltpu.sync_copy(dst_hbm.at[pl.ds(wid * _EPW + j * _DC, _DC)], g)
        for g in idx_gs:
            pltpu.sync_copy(ones_v, deg_s.at[g], add=True)

    def gstart(chunk, idxbuf, rbuf, sem):
        base = wid * _EPW + chunk * _EC
        pltpu.sync_copy(src_hbm.at[pl.ds(base, _EC)], idxbuf)
        pltpu.async_copy(h_hbm.at[idxbuf], rbuf, sem)

    def gwait(idxbuf, rbuf, sem):
        pltpu.make_async_copy(h_hbm.at[idxbuf], rbuf, sem).wait()

    def scat(chunk, idxbuf, rbuf):
        base = wid * _EPW + chunk * _EC
        pltpu.sync_copy(dst_hbm.at[pl.ds(base, _EC)], idxbuf)
        pltpu.sync_copy(rbuf, acc.at[idxbuf], add=True)

    gstart(0, idx_sa, rows_a, sem_a)

    def body(i, _):
        a = 2 * i
        gstart(a + 1, idx_sb, rows_b, sem_b)
        gwait(idx_sa, rows_a, sem_a)
        scat(a, idx_da, rows_a)
        gstart(a + 2, idx_sa, rows_a, sem_a)
        gwait(idx_sb, rows_b, sem_b)
        scat(a + 1, idx_db, rows_b)
        return 0
    lax.fori_loop(0, (_NCH - 1) // 2, body, 0)
    gwait(idx_sa, rows_a, sem_a)
    scat(_NCH - 1, idx_da, rows_a)

    plsc.subcore_barrier()

    pltpu.sync_copy(acc.at[pl.ds(r0, _RPT)], agg_out.at[c, pl.ds(r0, _RPT)])

    if with_deg:
        @pl.when(c == 0)
        def _():
            pltpu.sync_copy(deg_s.at[pl.ds(r0, _RPT)],
                            dega_out.at[pl.ds(r0, _RPT)])

        @pl.when(c == 1)
        def _():
            pltpu.sync_copy(deg_s.at[pl.ds(r0, _RPT)],
                            degb_out.at[pl.ds(r0, _RPT)])


def _agg_scratch():
    return [
        pltpu.VMEM_SHARED((_NP, _D), jnp.float32),
        pltpu.VMEM_SHARED((_NP,), jnp.float32),
        pltpu.VMEM((_EC,), jnp.int32),
        pltpu.VMEM((_EC,), jnp.int32),
        pltpu.VMEM((_EC,), jnp.int32),
        pltpu.VMEM((_EC,), jnp.int32),
        pltpu.VMEM((_DC,), jnp.int32),
        pltpu.VMEM((_DC,), jnp.int32),
        pltpu.VMEM((_DC,), jnp.int32),
        pltpu.VMEM((_DC,), jnp.int32),
        pltpu.VMEM((_DC,), jnp.int32),
        pltpu.VMEM((_EC, _D), jnp.float32),
        pltpu.VMEM((_EC, _D), jnp.float32),
        pltpu.VMEM((_DC,), jnp.float32),
        pltpu.SemaphoreType.DMA,
        pltpu.SemaphoreType.DMA,
    ]


_half_shape = jax.ShapeDtypeStruct((_NP, _HD), jnp.float32)

_sc_agg_deg = functools.partial(
    pl.kernel,
    out_type=[
        jax.ShapeDtypeStruct((_NC, _NP, _D), jnp.float32),
        jax.ShapeDtypeStruct((_NP,), jnp.float32),
        jax.ShapeDtypeStruct((_NP,), jnp.float32),
    ],
    mesh=plsc.VectorSubcoreMesh(core_axis_name="c", subcore_axis_name="s"),
    scratch_types=_agg_scratch(),
)(functools.partial(_sc_agg_body, True))

_sc_agg_nodeg = functools.partial(
    pl.kernel,
    out_type=jax.ShapeDtypeStruct((_NC, _NP, _D), jnp.float32),
    mesh=plsc.VectorSubcoreMesh(core_axis_name="c", subcore_axis_name="s"),
    scratch_types=_agg_scratch(),
)(functools.partial(_sc_agg_body, False))


def _sc_gather_body(h_hbm, ps_hbm, pd_hbm, ns_hbm, nd_hbm, bias_hbm,
                    sp_out, dp_out, sn_out, dn_out, bdiff_out,
                    rows, psv, pdv, nsv, ndv,
                    pbs_v, pbd_v, nbs_v, nbd_v, bd_v):
    c = lax.axis_index("c")
    s = lax.axis_index("s")
    wid = s * _NC + c
    base = wid * _PPW

    pltpu.sync_copy(ps_hbm.at[pl.ds(base, _PPW)], psv)
    pltpu.sync_copy(pd_hbm.at[pl.ds(base, _PPW)], pdv)
    pltpu.sync_copy(ns_hbm.at[pl.ds(base, _PPW)], nsv)
    pltpu.sync_copy(nd_hbm.at[pl.ds(base, _PPW)], ndv)

    pltpu.sync_copy(bias_hbm.at[psv], pbs_v)
    pltpu.sync_copy(bias_hbm.at[pdv], pbd_v)
    pltpu.sync_copy(bias_hbm.at[nsv], nbs_v)
    pltpu.sync_copy(bias_hbm.at[ndv], nbd_v)

    def comb(t, _):
        o = t * _L
        bd_v[pl.ds(o, _L)] = (nbs_v[pl.ds(o, _L)] + nbd_v[pl.ds(o, _L)]
                              - pbs_v[pl.ds(o, _L)] - pbd_v[pl.ds(o, _L)])
        return 0
    lax.fori_loop(0, _PPW // _L, comb, 0)
    pltpu.sync_copy(bd_v, bdiff_out.at[pl.ds(base, _PPW)])

    for idx_v, out in ((psv, sp_out), (pdv, dp_out),
                       (nsv, sn_out), (ndv, dn_out)):
        pltpu.sync_copy(h_hbm.at[idx_v], rows)
        pltpu.sync_copy(rows, out.at[pl.ds(base, _PPW)])


_sc_gather = functools.partial(
    pl.kernel,
    out_type=[
        jax.ShapeDtypeStruct((_P, _D), jnp.float32),
        jax.ShapeDtypeStruct((_P, _D), jnp.float32),
        jax.ShapeDtypeStruct((_P, _D), jnp.float32),
        jax.ShapeDtypeStruct((_P, _D), jnp.float32),
        jax.ShapeDtypeStruct((_P,), jnp.float32),
    ],
    mesh=plsc.VectorSubcoreMesh(core_axis_name="c", subcore_axis_name="s"),
    scratch_types=[
        pltpu.VMEM((_PPW, _D), jnp.float32),
        pltpu.VMEM((_PPW,), jnp.int32),
        pltpu.VMEM((_PPW,), jnp.int32),
        pltpu.VMEM((_PPW,), jnp.int32),
        pltpu.VMEM((_PPW,), jnp.int32),
        pltpu.VMEM((_PPW,), jnp.float32),
        pltpu.VMEM((_PPW,), jnp.float32),
        pltpu.VMEM((_PPW,), jnp.float32),
        pltpu.VMEM((_PPW,), jnp.float32),
        pltpu.VMEM((_PPW,), jnp.float32),
    ],
)(_sc_gather_body)


_BR = 1024


def _dot(a, b):
    return jnp.dot(a, b, preferred_element_type=jnp.float32,
                   precision=lax.Precision.HIGHEST)


def _tc_proj_body(x_ref, wp_ref, h0_ref):
    h0_ref[...] = _dot(x_ref[...], wp_ref[...])


def _tc_self_body(h_ref, ws_ref, s_ref):
    s_ref[...] = _dot(h_ref[...], ws_ref[...])


def _neigh_relu(selfp_ref, a_ref, b_ref, dega_ref, degb_ref, wn_ref,
                bias_ref):
    scale = 1.0 / jnp.maximum(dega_ref[...] + degb_ref[...], 1.0)
    agg = (a_ref[...] + b_ref[...]) * scale
    neigh = _dot(agg, wn_ref[...])
    return jnp.maximum(selfp_ref[...] + neigh + bias_ref[...], 0.0)


def _tc_layer_body(selfp_ref, a_ref, b_ref, dega_ref, degb_ref, wn_ref,
                   bias_ref, h_ref):
    h_ref[...] = _neigh_relu(selfp_ref, a_ref, b_ref, dega_ref, degb_ref,
                             wn_ref, bias_ref)


def _tc_final_body(selfp_ref, a_ref, b_ref, dega_ref, degb_ref, wn_ref,
                   bias_ref, h0_ref, out_ref):
    h = _neigh_relu(selfp_ref, a_ref, b_ref, dega_ref, degb_ref,
                    wn_ref, bias_ref)
    out_ref[...] = h0_ref[...] + h


def _tc_score_body(sp_ref, dp_ref, sn_ref, dn_ref, bd_ref, out_ref):
    posdot = jnp.sum(sp_ref[...] * dp_ref[...], axis=1, keepdims=True)
    negdot = jnp.sum(sn_ref[...] * dn_ref[...], axis=1, keepdims=True)
    out_ref[...] = jnp.maximum(negdot - posdot + bd_ref[...] + 1.0, 0.0)


def _row_spec():
    return pl.BlockSpec((_BR, _D), lambda i: (i, 0))


def _half_spec():
    return pl.BlockSpec((_BR, _HD), lambda i: (i, 0))


def _w_spec():
    return pl.BlockSpec((_D, _D), lambda i: (0, 0))


def _b_spec():
    return pl.BlockSpec((1, _D), lambda i: (0, 0))


def _deg_spec():
    return pl.BlockSpec((_BR, 1), lambda i: (i, 0))


_GRID = (_NP // _BR,)

_full_shape = jax.ShapeDtypeStruct((_NP, _D), jnp.float32)

_tc_proj = pl.pallas_call(
    _tc_proj_body,
    grid=_GRID,
    in_specs=[_row_spec(), _w_spec()],
    out_specs=_row_spec(),
    out_shape=_full_shape,
)

_tc_self = pl.pallas_call(
    _tc_self_body,
    grid=_GRID,
    in_specs=[_row_spec(), _w_spec()],
    out_specs=_row_spec(),
    out_shape=_full_shape,
)

_tc_layer = pl.pallas_call(
    _tc_layer_body,
    grid=_GRID,
    in_specs=[_row_spec(), _row_spec(), _row_spec(), _deg_spec(),
              _deg_spec(), _w_spec(), _b_spec()],
    out_specs=_row_spec(),
    out_shape=_full_shape,
)

_tc_final = pl.pallas_call(
    _tc_final_body,
    grid=_GRID,
    in_specs=[_row_spec(), _row_spec(), _row_spec(), _deg_spec(),
              _deg_spec(), _w_spec(), _b_spec(), _row_spec()],
    out_specs=_row_spec(),
    out_shape=_full_shape,
)

_tc_score = pl.pallas_call(
    _tc_score_body,
    grid=(_P // _BR,),
    in_specs=[_row_spec(), _row_spec(), _row_spec(), _row_spec(),
              _deg_spec()],
    out_specs=_deg_spec(),
    out_shape=jax.ShapeDtypeStruct((_P, 1), jnp.float32),
)


def kernel(x, edge_index, pos_edges, neg_edges, W_proj, W_self1, W_neigh1,
           b1, W_self2, W_neigh2, b2, item_bias):
    xp = jnp.pad(x, ((0, _NP - _N), (0, 0)))
    src = edge_index[0]
    dst = edge_index[1]

    h0 = _tc_proj(xp, W_proj)
    agg1, dega1, degb1 = _sc_agg_deg(h0, src, dst)
    s1 = _tc_self(h0, W_self1)
    dega = dega1[:, None]
    degb = degb1[:, None]
    h1 = _tc_layer(s1, agg1[0], agg1[1], dega, degb, W_neigh1,
                   b1.reshape(1, _D))
    agg2 = _sc_agg_nodeg(h1, src, dst)
    s2 = _tc_self(h1, W_self2)
    h_item = _tc_final(s2, agg2[0], agg2[1], dega, degb, W_neigh2,
                       b2.reshape(1, _D), h0)
    sp, dp, sn, dn, bdiff = _sc_gather(h_item, pos_edges[0], pos_edges[1],
                                       neg_edges[0], neg_edges[1], item_bias)
    return _tc_score(sp, dp, sn, dn, bdiff[:, None]).reshape(_P)
